# Initial kernel scaffold; baseline (speedup 1.0000x reference)
import functools

import jax
import jax.numpy as jnp
from jax import lax
from jax.experimental import pallas as pl
from jax.experimental.pallas import tpu as pltpu
from jax.experimental.pallas import tpu_sc as plsc

N_NODES = 10000
D = 128
N_EDGES = 320000
EPS = 1e-7
LN_EPS = 1e-5

NC = 2
NS = 16
NW = NC * NS
EW = N_EDGES // NW
K = 80
NCH = EW // K
RPT = N_NODES // NS
ZR = 125
CW = 16

BLK = 1000
GRID = N_NODES // BLK

_mesh = plsc.VectorSubcoreMesh(core_axis_name="c", subcore_axis_name="s")


def _sc_agg_body(with_cnt, xlin_hbm, src_hbm, dst_hbm, *rest):
    if with_cnt:
        (psum_hbm, cnt_hbm, sidx_v, didx_v, rows_v, zrow_v, ones_v, zcnt_v,
         acc_sh, cnt_sh) = rest
    else:
        (psum_hbm, sidx_v, didx_v, rows_v, zrow_v, acc_sh) = rest

    cid = lax.axis_index("c")
    sid = lax.axis_index("s")
    wid = sid * NC + cid

    zero16 = jnp.zeros((16,), jnp.float32)

    def zfill(i, _):
        zrow_v[i // (D // 16), pl.ds((i % (D // 16)) * 16, 16)] = zero16
        return 0
    lax.fori_loop(0, ZR * (D // 16), zfill, 0)

    def zcopy(j, _):
        pltpu.sync_copy(zrow_v, acc_sh.at[pl.ds(sid * RPT + j * ZR, ZR)])
        return 0
    lax.fori_loop(0, RPT // ZR, zcopy, 0)

    if with_cnt:
        one16 = jnp.full((16,), 1.0, jnp.float32)

        def ofill(i, _):
            ones_v[i, pl.ds(0, 16)] = one16
            zcnt_v[i, pl.ds(0, 16)] = zero16
            return 0
        lax.fori_loop(0, ZR, ofill, 0)

        def zccopy(j, _):
            pltpu.sync_copy(zcnt_v, cnt_sh.at[pl.ds(sid * RPT + j * ZR, ZR)])
            return 0
        lax.fori_loop(0, RPT // ZR, zccopy, 0)

    plsc.subcore_barrier()

    base = wid * EW

    def chunk(i, _):
        off = base + i * K
        pltpu.sync_copy(src_hbm.at[pl.ds(off, K)], sidx_v)
        pltpu.sync_copy(dst_hbm.at[pl.ds(off, K)], didx_v)
        pltpu.sync_copy(xlin_hbm.at[sidx_v], rows_v)
        pltpu.sync_copy(rows_v, acc_sh.at[didx_v], add=True)
        if with_cnt:
            pltpu.sync_copy(ones_v.at[pl.ds(0, K)], cnt_sh.at[didx_v], add=True)
        return 0
    lax.fori_loop(0, NCH, chunk, 0)

    plsc.subcore_barrier()

    pltpu.sync_copy(acc_sh.at[pl.ds(sid * RPT, RPT)],
                    psum_hbm.at[cid, pl.ds(sid * RPT, RPT)])
    if with_cnt:
        pltpu.sync_copy(cnt_sh.at[pl.ds(sid * RPT, RPT)],
                        cnt_hbm.at[cid, pl.ds(sid * RPT, RPT)])


def _make_sc_agg(with_cnt):
    out_type = [jax.ShapeDtypeStruct((NC, N_NODES, D), jnp.float32)]
    scratch = [
        pltpu.VMEM((K,), jnp.int32),
        pltpu.VMEM((K,), jnp.int32),
        pltpu.VMEM((K, D), jnp.float32),
        pltpu.VMEM((ZR, D), jnp.float32),
    ]
    if with_cnt:
        out_type.append(jax.ShapeDtypeStruct((NC, N_NODES, CW), jnp.float32))
        scratch += [
            pltpu.VMEM((ZR, CW), jnp.float32),
            pltpu.VMEM((ZR, CW), jnp.float32),
        ]
    scratch.append(pltpu.VMEM_SHARED((N_NODES, D), jnp.float32))
    if with_cnt:
        scratch.append(pltpu.VMEM_SHARED((N_NODES, CW), jnp.float32))
    return pl.kernel(
        functools.partial(_sc_agg_body, with_cnt),
        out_type=tuple(out_type) if with_cnt else out_type[0],
        mesh=_mesh,
        scratch_types=scratch,
    )


_sc_agg_cnt = _make_sc_agg(True)
_sc_agg = _make_sc_agg(False)


def _tc_pre_body(sq_ref, x_ref, w_ref, b_ref, xtan_ref, xlin_ref):
    sqrtc = sq_ref[0, 0]
    x = x_ref[...]
    col = lax.broadcasted_iota(jnp.int32, (BLK, D), 1)
    y = jnp.where(col == 0, 0.0, x)
    ynorm = jnp.maximum(jnp.sqrt(jnp.sum(y * y, axis=-1, keepdims=True)), EPS)
    xt = jnp.maximum(sqrtc * x[:, 0:1], 1.0 + EPS)
    theta = jnp.log(xt + jnp.sqrt(xt * xt - 1.0))
    xtan = (theta / (sqrtc * ynorm)) * y
    xtan_ref[...] = xtan
    xlin_ref[...] = lax.dot_general(
        xtan, w_ref[...], (((1,), (1,)), ((), ())),
        preferred_element_type=jnp.float32) + b_ref[...]


_tc_pre = pl.pallas_call(
    _tc_pre_body,
    grid=(GRID,),
    in_specs=[
        pl.BlockSpec(memory_space=pltpu.SMEM),
        pl.BlockSpec((BLK, D), lambda i: (i, 0)),
        pl.BlockSpec((D, D), lambda i: (0, 0)),
        pl.BlockSpec((1, D), lambda i: (0, 0)),
    ],
    out_specs=[
        pl.BlockSpec((BLK, D), lambda i: (i, 0)),
        pl.BlockSpec((BLK, D), lambda i: (i, 0)),
    ],
    out_shape=[
        jax.ShapeDtypeStruct((N_NODES, D), jnp.float32),
        jax.ShapeDtypeStruct((N_NODES, D), jnp.float32),
    ],
)


def _tc_post_body(sq_ref, xtan_ref, p0_ref, p1_ref, c0_ref, c1_ref,
                  g_ref, beta_ref, out_ref):
    sqrtc = sq_ref[0, 0]
    cnt = c0_ref[:, 0:1] + c1_ref[:, 0:1]
    agg = (p0_ref[...] + p1_ref[...]) / jnp.maximum(cnt, 1.0)
    h = xtan_ref[...] + agg
    mu = jnp.mean(h, axis=-1, keepdims=True)
    var = jnp.mean((h - mu) ** 2, axis=-1, keepdims=True)
    h = (h - mu) / jnp.sqrt(var + LN_EPS) * g_ref[...] + beta_ref[...]
    col = lax.broadcasted_iota(jnp.int32, (BLK, D), 1)
    u = jnp.where(col == 0, 0.0, h)
    norm = jnp.maximum(jnp.sqrt(jnp.sum(u * u, axis=-1, keepdims=True)), EPS)
    theta = sqrtc * norm
    e = jnp.exp(theta)
    einv = 1.0 / e
    time = (0.5 * (e + einv)) / sqrtc
    space = (0.5 * (e - einv)) * u / (sqrtc * norm)
    out_ref[...] = jnp.where(col == 0, time, space)


_tc_post = pl.pallas_call(
    _tc_post_body,
    grid=(GRID,),
    in_specs=[
        pl.BlockSpec(memory_space=pltpu.SMEM),
        pl.BlockSpec((BLK, D), lambda i: (i, 0)),
        pl.BlockSpec((BLK, D), lambda i: (i, 0)),
        pl.BlockSpec((BLK, D), lambda i: (i, 0)),
        pl.BlockSpec((BLK, CW), lambda i: (i, 0)),
        pl.BlockSpec((BLK, CW), lambda i: (i, 0)),
        pl.BlockSpec((1, D), lambda i: (0, 0)),
        pl.BlockSpec((1, D), lambda i: (0, 0)),
    ],
    out_specs=pl.BlockSpec((BLK, D), lambda i: (i, 0)),
    out_shape=jax.ShapeDtypeStruct((N_NODES, D), jnp.float32),
)


def kernel(x_hyp, edge_index, W0, b0, g0, beta0, c0,
           W1, b1, g1, beta1, c1):
    src = edge_index[0].astype(jnp.int32)
    dst = edge_index[1].astype(jnp.int32)
    sq0 = jnp.sqrt(jnp.clip(c0, 0.1, 10.0)).reshape(1, 1)
    sq1 = jnp.sqrt(jnp.clip(c1, 0.1, 10.0)).reshape(1, 1)
    b0r = b0.reshape(1, D)
    b1r = b1.reshape(1, D)
    g0r = g0.reshape(1, D)
    g1r = g1.reshape(1, D)
    beta0r = beta0.reshape(1, D)
    beta1r = beta1.reshape(1, D)

    xtan0, xlin0 = _tc_pre(sq0, x_hyp, W0, b0r)
    p, cnt = _sc_agg_cnt(xlin0, src, dst)
    x1 = _tc_post(sq0, xtan0, p[0], p[1], cnt[0], cnt[1], g0r, beta0r)

    xtan1, xlin1 = _tc_pre(sq1, x1, W1, b1r)
    p2 = _sc_agg(xlin1, src, dst)
    out = _tc_post(sq1, xtan1, p2[0], p2[1], cnt[0], cnt[1], g1r, beta1r)
    return out


# speedup vs baseline: 4.8610x; 4.8610x over previous
import functools

import jax
import jax.numpy as jnp
from jax import lax
from jax.experimental import pallas as pl
from jax.experimental.pallas import tpu as pltpu
from jax.experimental.pallas import tpu_sc as plsc

N_NODES = 10000
D = 128
N_EDGES = 320000
EPS = 1e-7
LN_EPS = 1e-5

NC = 2
NS = 16
NW = NC * NS
EW = N_EDGES // NW
K = 40
NCH = EW // K
N_PAD = 10240
RPT = N_PAD // NS
NSLOT = 5

BLK = 1024
GRID = N_PAD // BLK

_mesh = plsc.VectorSubcoreMesh(core_axis_name="c", subcore_axis_name="s")


def _sc_agg_body(with_cnt, xlin_hbm, src_hbm, dst_hbm, *rest):
    if with_cnt:
        (psum_hbm, cnt_hbm, sidx_v, didx_v, rows_v, ones_v,
         acc_sh, gsem, ssem) = rest
    else:
        (psum_hbm, sidx_v, didx_v, rows_v, ones_v,
         acc_sh, gsem, ssem) = rest

    cid = lax.axis_index("c")
    sid = lax.axis_index("s")
    wid = sid * NC + cid
    base = wid * EW

    zero16 = jnp.zeros((16,), jnp.float32)
    one16 = jnp.full((16,), 1.0, jnp.float32)

    def fill(buf, val16):
        def f(i, _):
            buf[i // (D // 16), pl.ds((i % (D // 16)) * 16, 16)] = val16
            return 0
        lax.fori_loop(0, K * (D // 16), f, 0)

    def zero_acc():
        fill(ones_v, zero16)

        def zcopy(j, _):
            pltpu.async_copy(
                ones_v, acc_sh.at[pl.ds(sid * RPT + j * K, K)], gsem.at[0])
            return 0
        lax.fori_loop(0, RPT // K, zcopy, 0)

        def zwait(j, _):
            pltpu.make_async_copy(
                ones_v, acc_sh.at[pl.ds(sid * RPT + j * K, K)],
                gsem.at[0]).wait()
            return 0
        lax.fori_loop(0, RPT // K, zwait, 0)

    def write_acc(out_hbm):
        def wround(j, _):
            for p in range(2):
                r = sid * RPT + (2 * j + p) * K

                @pl.when(j > 0)
                def _():
                    pltpu.make_async_copy(
                        rows_v.at[p], out_hbm.at[cid, pl.ds(r - 2 * K, K)],
                        gsem.at[p]).wait()
                pltpu.sync_copy(acc_sh.at[pl.ds(r, K)], rows_v.at[p])
                pltpu.async_copy(
                    rows_v.at[p], out_hbm.at[cid, pl.ds(r, K)], gsem.at[p])
            return 0
        nr = RPT // K // 2
        lax.fori_loop(0, nr, wround, 0)
        for p in range(2):
            r = sid * RPT + (2 * (nr - 1) + p) * K
            pltpu.make_async_copy(
                rows_v.at[p], out_hbm.at[cid, pl.ds(r, K)], gsem.at[p]).wait()

    zero_acc()
    plsc.subcore_barrier()

    for p in range(NSLOT):
        off = base + p * K
        pltpu.sync_copy(src_hbm.at[pl.ds(off, K)], sidx_v.at[p])
        pltpu.sync_copy(dst_hbm.at[pl.ds(off, K)], didx_v.at[p])
        pltpu.async_copy(xlin_hbm.at[sidx_v.at[p]], rows_v.at[p], gsem.at[p])

    def round_(j, _):
        for p in range(NSLOT):
            i = j * NSLOT + p
            pltpu.make_async_copy(
                xlin_hbm.at[sidx_v.at[p]], rows_v.at[p], gsem.at[p]).wait()
            pltpu.async_copy(
                rows_v.at[p], acc_sh.at[didx_v.at[p]], ssem.at[p], add=True)

            @pl.when(j < NCH // NSLOT - 1)
            def _():
                pltpu.make_async_copy(
                    rows_v.at[p], acc_sh.at[didx_v.at[p]], ssem.at[p]).wait()
                off = base + (i + NSLOT) * K
                pltpu.sync_copy(src_hbm.at[pl.ds(off, K)], sidx_v.at[p])
                pltpu.sync_copy(dst_hbm.at[pl.ds(off, K)], didx_v.at[p])
                pltpu.async_copy(
                    xlin_hbm.at[sidx_v.at[p]], rows_v.at[p], gsem.at[p])
        return 0
    lax.fori_loop(0, NCH // NSLOT, round_, 0)

    for p in range(NSLOT):
        pltpu.make_async_copy(
            rows_v.at[p], acc_sh.at[didx_v.at[p]], ssem.at[p]).wait()

    plsc.subcore_barrier()
    write_acc(psum_hbm)

    if with_cnt:
        plsc.subcore_barrier()
        zero_acc()
        plsc.subcore_barrier()
        fill(ones_v, one16)

        for p in range(NSLOT):
            pltpu.sync_copy(dst_hbm.at[pl.ds(base + p * K, K)], didx_v.at[p])
            pltpu.async_copy(
                ones_v, acc_sh.at[didx_v.at[p]], ssem.at[p], add=True)

        def cround(j, _):
            for p in range(NSLOT):
                i = j * NSLOT + p
                pltpu.make_async_copy(
                    ones_v, acc_sh.at[didx_v.at[p]], ssem.at[p]).wait()

                @pl.when(j < NCH // NSLOT - 1)
                def _():
                    off = base + (i + NSLOT) * K
                    pltpu.sync_copy(dst_hbm.at[pl.ds(off, K)], didx_v.at[p])
                    pltpu.async_copy(
                        ones_v, acc_sh.at[didx_v.at[p]], ssem.at[p], add=True)
            return 0
        lax.fori_loop(0, NCH // NSLOT, cround, 0)

        plsc.subcore_barrier()
        write_acc(cnt_hbm)


def _make_sc_agg(with_cnt):
    out_type = [jax.ShapeDtypeStruct((NC, N_PAD, D), jnp.float32)]
    scratch = [
        pltpu.VMEM((NSLOT, K), jnp.int32),
        pltpu.VMEM((NSLOT, K), jnp.int32),
        pltpu.VMEM((NSLOT, K, D), jnp.float32),
        pltpu.VMEM((K, D), jnp.float32),
    ]
    if with_cnt:
        out_type.append(jax.ShapeDtypeStruct((NC, N_PAD, D), jnp.float32))
    scratch.append(pltpu.VMEM_SHARED((N_PAD, D), jnp.float32))
    scratch.append(pltpu.SemaphoreType.DMA((NSLOT,)))
    scratch.append(pltpu.SemaphoreType.DMA((NSLOT,)))
    return pl.kernel(
        functools.partial(_sc_agg_body, with_cnt),
        out_type=tuple(out_type) if with_cnt else out_type[0],
        mesh=_mesh,
        scratch_types=scratch,
    )


_sc_agg_cnt = _make_sc_agg(True)
_sc_agg = _make_sc_agg(False)


def _tc_pre_body(sq_ref, x_ref, w_ref, b_ref, xtan_ref, xlin_ref):
    sqrtc = sq_ref[0, 0]
    x = x_ref[...]
    col = lax.broadcasted_iota(jnp.int32, (BLK, D), 1)
    y = jnp.where(col == 0, 0.0, x)
    ynorm = jnp.maximum(jnp.sqrt(jnp.sum(y * y, axis=-1, keepdims=True)), EPS)
    xt = jnp.maximum(sqrtc * x[:, 0:1], 1.0 + EPS)
    theta = jnp.log(xt + jnp.sqrt(xt * xt - 1.0))
    xtan = (theta / (sqrtc * ynorm)) * y
    xtan_ref[...] = xtan
    xlin_ref[...] = lax.dot_general(
        xtan, w_ref[...], (((1,), (1,)), ((), ())),
        preferred_element_type=jnp.float32) + b_ref[...]


_tc_pre = pl.pallas_call(
    _tc_pre_body,
    grid=(GRID,),
    in_specs=[
        pl.BlockSpec(memory_space=pltpu.SMEM),
        pl.BlockSpec((BLK, D), lambda i: (i, 0)),
        pl.BlockSpec((D, D), lambda i: (0, 0)),
        pl.BlockSpec((1, D), lambda i: (0, 0)),
    ],
    out_specs=[
        pl.BlockSpec((BLK, D), lambda i: (i, 0)),
        pl.BlockSpec((BLK, D), lambda i: (i, 0)),
    ],
    out_shape=[
        jax.ShapeDtypeStruct((N_PAD, D), jnp.float32),
        jax.ShapeDtypeStruct((N_PAD, D), jnp.float32),
    ],
)


def _tc_post_body(sq_ref, xtan_ref, p0_ref, p1_ref, c0_ref, c1_ref,
                  g_ref, beta_ref, out_ref):
    sqrtc = sq_ref[0, 0]
    cnt = c0_ref[...] + c1_ref[...]
    agg = (p0_ref[...] + p1_ref[...]) / jnp.maximum(cnt, 1.0)
    h = xtan_ref[...] + agg
    mu = jnp.mean(h, axis=-1, keepdims=True)
    var = jnp.mean((h - mu) ** 2, axis=-1, keepdims=True)
    h = (h - mu) / jnp.sqrt(var + LN_EPS) * g_ref[...] + beta_ref[...]
    col = lax.broadcasted_iota(jnp.int32, (BLK, D), 1)
    u = jnp.where(col == 0, 0.0, h)
    norm = jnp.maximum(jnp.sqrt(jnp.sum(u * u, axis=-1, keepdims=True)), EPS)
    theta = sqrtc * norm
    e = jnp.exp(theta)
    einv = 1.0 / e
    time = (0.5 * (e + einv)) / sqrtc
    space = (0.5 * (e - einv)) * u / (sqrtc * norm)
    out_ref[...] = jnp.where(col == 0, time, space)


_tc_post = pl.pallas_call(
    _tc_post_body,
    grid=(GRID,),
    in_specs=[
        pl.BlockSpec(memory_space=pltpu.SMEM),
        pl.BlockSpec((BLK, D), lambda i: (i, 0)),
        pl.BlockSpec((BLK, D), lambda i: (i, 0)),
        pl.BlockSpec((BLK, D), lambda i: (i, 0)),
        pl.BlockSpec((BLK, D), lambda i: (i, 0)),
        pl.BlockSpec((BLK, D), lambda i: (i, 0)),
        pl.BlockSpec((1, D), lambda i: (0, 0)),
        pl.BlockSpec((1, D), lambda i: (0, 0)),
    ],
    out_specs=pl.BlockSpec((BLK, D), lambda i: (i, 0)),
    out_shape=jax.ShapeDtypeStruct((N_PAD, D), jnp.float32),
)


def kernel(x_hyp, edge_index, W0, b0, g0, beta0, c0,
           W1, b1, g1, beta1, c1):
    src = edge_index[0].astype(jnp.int32)
    dst = edge_index[1].astype(jnp.int32)
    sq0 = jnp.sqrt(jnp.clip(c0, 0.1, 10.0)).reshape(1, 1)
    sq1 = jnp.sqrt(jnp.clip(c1, 0.1, 10.0)).reshape(1, 1)
    b0r = b0.reshape(1, D)
    b1r = b1.reshape(1, D)
    g0r = g0.reshape(1, D)
    g1r = g1.reshape(1, D)
    beta0r = beta0.reshape(1, D)
    beta1r = beta1.reshape(1, D)
    xp = jnp.pad(x_hyp, ((0, N_PAD - N_NODES), (0, 0)))

    xtan0, xlin0 = _tc_pre(sq0, xp, W0, b0r)
    p, cnt = _sc_agg_cnt(xlin0, src, dst)
    x1 = _tc_post(sq0, xtan0, p[0], p[1], cnt[0], cnt[1], g0r, beta0r)

    xtan1, xlin1 = _tc_pre(sq1, x1, W1, b1r)
    p2 = _sc_agg(xlin1, src, dst)
    out = _tc_post(sq1, xtan1, p2[0], p2[1], cnt[0], cnt[1], g1r, beta1r)
    return out[:N_NODES]
